# Initial kernel scaffold; baseline (speedup 1.0000x reference)
import jax
import jax.numpy as jnp
from jax.experimental import pallas as pl


def kernel(x, edge_index, batch, W1, b1, W2, b2, Wlin, blin):
    raise NotImplementedError("write your pallas kernel here")


# speedup vs baseline: 53.1262x; 53.1262x over previous
import functools

import jax
import jax.numpy as jnp
from jax import lax
from jax.experimental import pallas as pl
from jax.experimental.pallas import tpu as pltpu
from jax.experimental.pallas import tpu_sc as plsc

NC = 2
NS = 16
NW = NC * NS
LN = 128
BL = 8
G_OUT = 128


def _worker_rows(wid, total_rows):
    ngrp = total_rows // 8
    base = ngrp // NW
    rem = ngrp % NW
    grps = jnp.where(wid < rem, base + 1, base)
    grp0 = jnp.where(wid < rem, wid * (base + 1),
                     rem * (base + 1) + (wid - rem) * base)
    return grp0 * 8, grps * 8


def _deg_body(dst_hbm, zeros_hbm, out_hbm, acc_sh, idx_v, ones_v, bnc_v, sem_s):
    c = lax.axis_index("c")
    s = lax.axis_index("s")
    wid = s * NC + c
    npad = acc_sh.shape[0]
    sl = npad // NS
    bc = zeros_hbm.shape[0]
    nq = sl // bc

    @pl.loop(0, LN // 16)
    def _(i):
        ones_v[pl.ds(i * 16, 16)] = jnp.ones((16,), jnp.float32)

    pltpu.sync_copy(zeros_hbm, bnc_v)
    off = pl.multiple_of(s * sl, 8)
    for q in range(nq):
        pltpu.sync_copy(bnc_v, acc_sh.at[pl.ds(pl.multiple_of(off + q * bc, 8), bc)])
    plsc.subcore_barrier()

    row0, rows = _worker_rows(wid, dst_hbm.shape[0])
    nb = rows // BL

    @pl.loop(0, nb)
    def _(g):
        r0 = pl.multiple_of(row0 + g * BL, 8)
        pltpu.sync_copy(dst_hbm.at[pl.ds(r0, BL)], idx_v)
        for j in range(BL):
            pltpu.async_copy(ones_v, acc_sh.at[idx_v.at[j]], sem_s, add=True)
        for j in range(BL):
            pltpu.make_async_copy(ones_v, acc_sh.at[idx_v.at[j]], sem_s).wait()

    plsc.subcore_barrier()
    for q in range(nq):
        qo = pl.multiple_of(off + q * bc, 8)
        oo = pl.multiple_of(c * npad + s * sl + q * bc, 8)
        pltpu.sync_copy(acc_sh.at[pl.ds(qo, bc)], bnc_v)
        pltpu.sync_copy(bnc_v, out_hbm.at[pl.ds(oo, bc)])


def _edge_body(src_hbm, dst_hbm, tab_hbm, zeros_hbm, out_hbm,
               acc_sh, sidx_v, didx_v, val_v, bnc_v, sem_g, sem_s):
    c = lax.axis_index("c")
    s = lax.axis_index("s")
    wid = s * NC + c
    npad = acc_sh.shape[0]
    sl = npad // NS
    bc = zeros_hbm.shape[0]
    nq = sl // bc

    pltpu.sync_copy(zeros_hbm, bnc_v)
    off = pl.multiple_of(s * sl, 8)
    for q in range(nq):
        pltpu.sync_copy(bnc_v, acc_sh.at[pl.ds(pl.multiple_of(off + q * bc, 8), bc)])
    plsc.subcore_barrier()

    row0, rows = _worker_rows(wid, src_hbm.shape[0])
    nb = rows // BL

    @pl.loop(0, nb)
    def _(g):
        r0 = pl.multiple_of(row0 + g * BL, 8)
        pltpu.sync_copy(src_hbm.at[pl.ds(r0, BL)], sidx_v)
        pltpu.sync_copy(dst_hbm.at[pl.ds(r0, BL)], didx_v)
        for j in range(BL):
            pltpu.async_copy(tab_hbm.at[sidx_v.at[j]], val_v.at[j], sem_g)
        for j in range(BL):
            pltpu.make_async_copy(tab_hbm.at[sidx_v.at[j]], val_v.at[j], sem_g).wait()
            pltpu.async_copy(val_v.at[j], acc_sh.at[didx_v.at[j]], sem_s, add=True)
        for j in range(BL):
            pltpu.make_async_copy(val_v.at[j], acc_sh.at[didx_v.at[j]], sem_s).wait()

    plsc.subcore_barrier()
    for q in range(nq):
        qo = pl.multiple_of(off + q * bc, 8)
        oo = pl.multiple_of(c * npad + s * sl + q * bc, 8)
        pltpu.sync_copy(acc_sh.at[pl.ds(qo, bc)], bnc_v)
        pltpu.sync_copy(bnc_v, out_hbm.at[pl.ds(oo, bc)])


def _sc_deg(dst_rows, npad):
    sl = npad // NS
    bc = max(v for v in range(8, 513, 8) if sl % v == 0)
    mesh = plsc.VectorSubcoreMesh(core_axis_name="c", subcore_axis_name="s")
    zeros = jnp.zeros((bc,), jnp.float32)
    return pl.kernel(
        _deg_body,
        out_type=jax.ShapeDtypeStruct((NC * npad,), jnp.float32),
        mesh=mesh,
        compiler_params=pltpu.CompilerParams(use_tc_tiling_on_sc=False),
        scratch_types=[
            pltpu.VMEM_SHARED((npad,), jnp.float32),
            pltpu.VMEM((BL, LN), jnp.int32),
            pltpu.VMEM((LN,), jnp.float32),
            pltpu.VMEM((bc,), jnp.float32),
            pltpu.SemaphoreType.DMA,
        ],
    )(dst_rows, zeros)


def _sc_edge(src_rows, dst_rows, table, npad, d):
    sl = npad // NS
    bc = max(v for v in range(8, 513, 8) if sl % v == 0)
    mesh = plsc.VectorSubcoreMesh(core_axis_name="c", subcore_axis_name="s")
    zeros = jnp.zeros((bc, d), jnp.float32)
    return pl.kernel(
        _edge_body,
        out_type=jax.ShapeDtypeStruct((NC * npad, d), jnp.float32),
        mesh=mesh,
        compiler_params=pltpu.CompilerParams(use_tc_tiling_on_sc=False),
        scratch_types=[
            pltpu.VMEM_SHARED((npad, d), jnp.float32),
            pltpu.VMEM((BL, LN), jnp.int32),
            pltpu.VMEM((BL, LN), jnp.int32),
            pltpu.VMEM((BL, LN, d), jnp.float32),
            pltpu.VMEM((bc, d), jnp.float32),
            pltpu.SemaphoreType.DMA,
            pltpu.SemaphoreType.DMA,
        ],
    )(src_rows, dst_rows, table, zeros)


def _prep_body(deg0_ref, deg1_ref, x_ref, dinv_ref, u_ref):
    deg = deg0_ref[...] + deg1_ref[...] + 1.0
    dinv = lax.rsqrt(deg)
    dinv_ref[...] = dinv
    xz = jnp.concatenate(
        [x_ref[...], jnp.zeros((x_ref.shape[0], 14), jnp.float32)], axis=1)
    u_ref[...] = xz * dinv


def _l1_body(s0_ref, s1_ref, u_ref, dinv_ref, w1_ref, b1_ref, p_ref):
    ssum = s0_ref[...] + s1_ref[...] + u_ref[...]
    t = ssum[:, 0:1] * w1_ref[0:1, :] + ssum[:, 1:2] * w1_ref[1:2, :]
    dinv = dinv_ref[...]
    h1 = jnp.tanh(dinv * t + b1_ref[...][None, :])
    p_ref[...] = h1 * dinv


def _l2_pool_body(s0_ref, s1_ref, p_ref, dinv_ref, batch_ref, w2_ref, b2_ref,
                  wlin_ref, blin_ref, out_ref, acc_ref):
    i = pl.program_id(0)
    m = s0_ref[...] + s1_ref[...] + p_ref[...]
    z = jnp.dot(m, w2_ref[...], preferred_element_type=jnp.float32)
    h2 = jnp.tanh(dinv_ref[...] * z + b2_ref[...][None, :])
    gid = lax.broadcasted_iota(jnp.int32, (1, G_OUT), 1)
    oh = (batch_ref[...] == gid).astype(jnp.float32)
    hcat = jnp.concatenate(
        [h2, jnp.ones((h2.shape[0], 1), jnp.float32)], axis=1)
    part = lax.dot_general(oh, hcat, (((0,), (0,)), ((), ())),
                           preferred_element_type=jnp.float32)

    @pl.when(i == 0)
    def _():
        acc_ref[...] = part

    @pl.when(i > 0)
    def _():
        acc_ref[...] += part

    @pl.when(i == pl.num_programs(0) - 1)
    def _():
        pooled = acc_ref[...][:, :16] / jnp.maximum(acc_ref[...][:, 16:17], 1.0)
        out_ref[...] = (jnp.dot(pooled, wlin_ref[...],
                                preferred_element_type=jnp.float32)
                        + blin_ref[...][None, :])


def kernel(x, edge_index, batch, W1, b1, W2, b2, Wlin, blin):
    n = x.shape[0]
    e = edge_index.shape[1]
    assert e % LN == 0
    r = e // LN
    npad = ((n + LN - 1) // LN) * LN
    bn = npad // 16
    grid = 16

    assert r % 8 == 0
    src_rows = edge_index[0].reshape(r, LN)
    dst_rows = edge_index[1].reshape(r, LN)
    x_pad = jnp.concatenate(
        [x, jnp.zeros((npad - n, x.shape[1]), x.dtype)], axis=0)
    batch_pad = jnp.concatenate(
        [batch, jnp.full((npad - n,), G_OUT, batch.dtype)], axis=0).reshape(npad, 1)

    deg2 = _sc_deg(dst_rows, npad).reshape(NC, npad)

    dinv, u = pl.pallas_call(
        _prep_body,
        grid=(grid,),
        in_specs=[
            pl.BlockSpec((bn, 1), lambda i: (i, 0)),
            pl.BlockSpec((bn, 1), lambda i: (i, 0)),
            pl.BlockSpec((bn, 2), lambda i: (i, 0)),
        ],
        out_specs=[
            pl.BlockSpec((bn, 1), lambda i: (i, 0)),
            pl.BlockSpec((bn, 16), lambda i: (i, 0)),
        ],
        out_shape=[
            jax.ShapeDtypeStruct((npad, 1), jnp.float32),
            jax.ShapeDtypeStruct((npad, 16), jnp.float32),
        ],
    )(deg2[0].reshape(npad, 1), deg2[1].reshape(npad, 1), x_pad)

    s1 = _sc_edge(src_rows, dst_rows, u, npad, 16).reshape(NC, npad, 16)

    p = pl.pallas_call(
        _l1_body,
        grid=(grid,),
        in_specs=[
            pl.BlockSpec((bn, 16), lambda i: (i, 0)),
            pl.BlockSpec((bn, 16), lambda i: (i, 0)),
            pl.BlockSpec((bn, 16), lambda i: (i, 0)),
            pl.BlockSpec((bn, 1), lambda i: (i, 0)),
            pl.BlockSpec((2, 16), lambda i: (0, 0)),
            pl.BlockSpec((16,), lambda i: (0,)),
        ],
        out_specs=pl.BlockSpec((bn, 16), lambda i: (i, 0)),
        out_shape=jax.ShapeDtypeStruct((npad, 16), jnp.float32),
    )(s1[0], s1[1], u, dinv, W1, b1)

    s2 = _sc_edge(src_rows, dst_rows, p, npad, 16).reshape(NC, npad, 16)

    out = pl.pallas_call(
        _l2_pool_body,
        grid=(grid,),
        in_specs=[
            pl.BlockSpec((bn, 16), lambda i: (i, 0)),
            pl.BlockSpec((bn, 16), lambda i: (i, 0)),
            pl.BlockSpec((bn, 16), lambda i: (i, 0)),
            pl.BlockSpec((bn, 1), lambda i: (i, 0)),
            pl.BlockSpec((bn, 1), lambda i: (i, 0)),
            pl.BlockSpec((16, 16), lambda i: (0, 0)),
            pl.BlockSpec((16,), lambda i: (0,)),
            pl.BlockSpec((16, 1), lambda i: (0, 0)),
            pl.BlockSpec((1,), lambda i: (0,)),
        ],
        out_specs=pl.BlockSpec((G_OUT, 1), lambda i: (0, 0)),
        out_shape=jax.ShapeDtypeStruct((G_OUT, 1), jnp.float32),
        scratch_shapes=[pltpu.VMEM((G_OUT, 17), jnp.float32)],
    )(s2[0], s2[1], p, dinv, batch_pad, W2, b2, Wlin, blin)

    return out


# speedup vs baseline: 76.9895x; 1.4492x over previous
import functools

import jax
import jax.numpy as jnp
from jax import lax
from jax.experimental import pallas as pl
from jax.experimental.pallas import tpu as pltpu
from jax.experimental.pallas import tpu_sc as plsc

NC = 2
NS = 16
NW = NC * NS
LN = 128
BL = 8
G_OUT = 128


def _worker_rows(wid, total_rows):
    ngrp = total_rows // 8
    base = ngrp // NW
    rem = ngrp % NW
    grps = jnp.where(wid < rem, base + 1, base)
    grp0 = jnp.where(wid < rem, wid * (base + 1),
                     rem * (base + 1) + (wid - rem) * base)
    return grp0 * 8, grps * 8


def _deg_body(dst_hbm, zeros_hbm, out_hbm, acc_sh, idx_v, ones_v, bnc_v,
              sem_i, sem_s):
    c = lax.axis_index("c")
    s = lax.axis_index("s")
    wid = s * NC + c
    npad = acc_sh.shape[0]
    sl = npad // NS
    bc = zeros_hbm.shape[0]
    nq = sl // bc

    @pl.loop(0, LN // 16)
    def _(i):
        ones_v[pl.ds(i * 16, 16)] = jnp.ones((16,), jnp.float32)

    pltpu.sync_copy(zeros_hbm, bnc_v)
    off = pl.multiple_of(s * sl, 8)
    for q in range(nq):
        pltpu.sync_copy(bnc_v, acc_sh.at[pl.ds(pl.multiple_of(off + q * bc, 8), bc)])
    plsc.subcore_barrier()

    row0, rows = _worker_rows(wid, dst_hbm.shape[0])
    nb = rows // BL

    pltpu.async_copy(dst_hbm.at[pl.ds(row0, BL)], idx_v.at[0], sem_i)

    @pl.loop(0, nb)
    def _(g):
        par = g % 2
        r0 = pl.multiple_of(row0 + g * BL, 8)
        pltpu.make_async_copy(dst_hbm.at[pl.ds(r0, BL)], idx_v.at[par], sem_i).wait()

        @pl.when(g + 1 < nb)
        def _():
            rn = pl.multiple_of(row0 + (g + 1) * BL, 8)
            pltpu.async_copy(dst_hbm.at[pl.ds(rn, BL)], idx_v.at[1 - par], sem_i)

        for j in range(BL):
            pltpu.async_copy(ones_v, acc_sh.at[idx_v.at[par, j]], sem_s, add=True)
        for j in range(BL):
            pltpu.make_async_copy(ones_v, acc_sh.at[idx_v.at[par, j]], sem_s).wait()

    plsc.subcore_barrier()
    for q in range(nq):
        qo = pl.multiple_of(off + q * bc, 8)
        oo = pl.multiple_of(c * npad + s * sl + q * bc, 8)
        pltpu.sync_copy(acc_sh.at[pl.ds(qo, bc)], bnc_v)
        pltpu.sync_copy(bnc_v, out_hbm.at[pl.ds(oo, bc)])


def _edge_body(src_hbm, dst_hbm, tab_hbm, zeros_hbm, out_hbm,
               acc_sh, sidx_v, didx_v, val_v, bnc_v, sem_i, sem_g, sem_s):
    c = lax.axis_index("c")
    s = lax.axis_index("s")
    wid = s * NC + c
    npad = acc_sh.shape[0]
    sl = npad // NS
    bc = zeros_hbm.shape[0]
    nq = sl // bc

    pltpu.sync_copy(zeros_hbm, bnc_v)
    off = pl.multiple_of(s * sl, 8)
    for q in range(nq):
        pltpu.sync_copy(bnc_v, acc_sh.at[pl.ds(pl.multiple_of(off + q * bc, 8), bc)])
    plsc.subcore_barrier()

    row0, rows = _worker_rows(wid, src_hbm.shape[0])
    nb = rows // BL

    pltpu.async_copy(src_hbm.at[pl.ds(row0, BL)], sidx_v.at[0], sem_i)
    pltpu.async_copy(dst_hbm.at[pl.ds(row0, BL)], didx_v.at[0], sem_i)

    @pl.loop(0, nb)
    def _(g):
        par = g % 2
        r0 = pl.multiple_of(row0 + g * BL, 8)
        pltpu.make_async_copy(src_hbm.at[pl.ds(r0, BL)], sidx_v.at[par], sem_i).wait()
        pltpu.make_async_copy(dst_hbm.at[pl.ds(r0, BL)], didx_v.at[par], sem_i).wait()
        for j in range(BL):
            pltpu.async_copy(tab_hbm.at[sidx_v.at[par, j]], val_v.at[j], sem_g)

        @pl.when(g + 1 < nb)
        def _():
            rn = pl.multiple_of(row0 + (g + 1) * BL, 8)
            pltpu.async_copy(src_hbm.at[pl.ds(rn, BL)], sidx_v.at[1 - par], sem_i)
            pltpu.async_copy(dst_hbm.at[pl.ds(rn, BL)], didx_v.at[1 - par], sem_i)

        for j in range(BL):
            pltpu.make_async_copy(tab_hbm.at[sidx_v.at[par, j]], val_v.at[j], sem_g).wait()
            pltpu.async_copy(val_v.at[j], acc_sh.at[didx_v.at[par, j]], sem_s, add=True)
        for j in range(BL):
            pltpu.make_async_copy(val_v.at[j], acc_sh.at[didx_v.at[par, j]], sem_s).wait()

    plsc.subcore_barrier()
    for q in range(nq):
        qo = pl.multiple_of(off + q * bc, 8)
        oo = pl.multiple_of(c * npad + s * sl + q * bc, 8)
        pltpu.sync_copy(acc_sh.at[pl.ds(qo, bc)], bnc_v)
        pltpu.sync_copy(bnc_v, out_hbm.at[pl.ds(oo, bc)])


def _sc_deg(dst_rows, npad):
    sl = npad // NS
    bc = max(v for v in range(8, 513, 8) if sl % v == 0)
    mesh = plsc.VectorSubcoreMesh(core_axis_name="c", subcore_axis_name="s")
    zeros = jnp.zeros((bc,), jnp.float32)
    return pl.kernel(
        _deg_body,
        out_type=jax.ShapeDtypeStruct((NC * npad,), jnp.float32),
        mesh=mesh,
        compiler_params=pltpu.CompilerParams(use_tc_tiling_on_sc=False),
        scratch_types=[
            pltpu.VMEM_SHARED((npad,), jnp.float32),
            pltpu.VMEM((2, BL, LN), jnp.int32),
            pltpu.VMEM((LN,), jnp.float32),
            pltpu.VMEM((bc,), jnp.float32),
            pltpu.SemaphoreType.DMA,
            pltpu.SemaphoreType.DMA,
        ],
    )(dst_rows, zeros)


def _sc_edge(src_rows, dst_rows, table, npad, d):
    sl = npad // NS
    bc = max(v for v in range(8, 513, 8) if sl % v == 0)
    mesh = plsc.VectorSubcoreMesh(core_axis_name="c", subcore_axis_name="s")
    zeros = jnp.zeros((bc, d), jnp.float32)
    return pl.kernel(
        _edge_body,
        out_type=jax.ShapeDtypeStruct((NC * npad, d), jnp.float32),
        mesh=mesh,
        compiler_params=pltpu.CompilerParams(use_tc_tiling_on_sc=False),
        scratch_types=[
            pltpu.VMEM_SHARED((npad, d), jnp.float32),
            pltpu.VMEM((2, BL, LN), jnp.int32),
            pltpu.VMEM((2, BL, LN), jnp.int32),
            pltpu.VMEM((BL, LN, d), jnp.float32),
            pltpu.VMEM((bc, d), jnp.float32),
            pltpu.SemaphoreType.DMA,
            pltpu.SemaphoreType.DMA,
            pltpu.SemaphoreType.DMA,
        ],
    )(src_rows, dst_rows, table, zeros)


def _prep_body(deg0_ref, deg1_ref, x_ref, dinv_ref, u_ref):
    deg = deg0_ref[...] + deg1_ref[...] + 1.0
    dinv = lax.rsqrt(deg)
    dinv_ref[...] = dinv
    xz = jnp.concatenate(
        [x_ref[...], jnp.zeros((x_ref.shape[0], 14), jnp.float32)], axis=1)
    u_ref[...] = xz * dinv


def _l1_body(s0_ref, s1_ref, u_ref, dinv_ref, w1_ref, b1_ref, p_ref):
    ssum = s0_ref[...] + s1_ref[...] + u_ref[...]
    t = ssum[:, 0:1] * w1_ref[0:1, :] + ssum[:, 1:2] * w1_ref[1:2, :]
    dinv = dinv_ref[...]
    h1 = jnp.tanh(dinv * t + b1_ref[...][None, :])
    p_ref[...] = h1 * dinv


def _l2_pool_body(s0_ref, s1_ref, p_ref, dinv_ref, batch_ref, w2_ref, b2_ref,
                  wlin_ref, blin_ref, out_ref, acc_ref):
    i = pl.program_id(0)
    m = s0_ref[...] + s1_ref[...] + p_ref[...]
    z = jnp.dot(m, w2_ref[...], preferred_element_type=jnp.float32)
    h2 = jnp.tanh(dinv_ref[...] * z + b2_ref[...][None, :])
    gid = lax.broadcasted_iota(jnp.int32, (1, G_OUT), 1)
    oh = (batch_ref[...] == gid).astype(jnp.float32)
    hcat = jnp.concatenate(
        [h2, jnp.ones((h2.shape[0], 1), jnp.float32)], axis=1)
    part = lax.dot_general(oh, hcat, (((0,), (0,)), ((), ())),
                           preferred_element_type=jnp.float32)

    @pl.when(i == 0)
    def _():
        acc_ref[...] = part

    @pl.when(i > 0)
    def _():
        acc_ref[...] += part

    @pl.when(i == pl.num_programs(0) - 1)
    def _():
        pooled = acc_ref[...][:, :16] / jnp.maximum(acc_ref[...][:, 16:17], 1.0)
        out_ref[...] = (jnp.dot(pooled, wlin_ref[...],
                                preferred_element_type=jnp.float32)
                        + blin_ref[...][None, :])


def kernel(x, edge_index, batch, W1, b1, W2, b2, Wlin, blin):
    n = x.shape[0]
    e = edge_index.shape[1]
    assert e % LN == 0
    r = e // LN
    npad = ((n + LN - 1) // LN) * LN
    bn = npad // 16
    grid = 16

    assert r % 8 == 0
    src_rows = edge_index[0].reshape(r, LN)
    dst_rows = edge_index[1].reshape(r, LN)
    x_pad = jnp.concatenate(
        [x, jnp.zeros((npad - n, x.shape[1]), x.dtype)], axis=0)
    batch_pad = jnp.concatenate(
        [batch, jnp.full((npad - n,), G_OUT, batch.dtype)], axis=0).reshape(npad, 1)

    deg2 = _sc_deg(dst_rows, npad).reshape(NC * npad, 1)

    dinv, u = pl.pallas_call(
        _prep_body,
        grid=(grid,),
        in_specs=[
            pl.BlockSpec((bn, 1), lambda i: (i, 0)),
            pl.BlockSpec((bn, 1), lambda i: (i + grid, 0)),
            pl.BlockSpec((bn, 2), lambda i: (i, 0)),
        ],
        out_specs=[
            pl.BlockSpec((bn, 1), lambda i: (i, 0)),
            pl.BlockSpec((bn, 16), lambda i: (i, 0)),
        ],
        out_shape=[
            jax.ShapeDtypeStruct((npad, 1), jnp.float32),
            jax.ShapeDtypeStruct((npad, 16), jnp.float32),
        ],
    )(deg2, deg2, x_pad)

    s1 = _sc_edge(src_rows, dst_rows, u, npad, 16)

    p = pl.pallas_call(
        _l1_body,
        grid=(grid,),
        in_specs=[
            pl.BlockSpec((bn, 16), lambda i: (i, 0)),
            pl.BlockSpec((bn, 16), lambda i: (i + grid, 0)),
            pl.BlockSpec((bn, 16), lambda i: (i, 0)),
            pl.BlockSpec((bn, 1), lambda i: (i, 0)),
            pl.BlockSpec((2, 16), lambda i: (0, 0)),
            pl.BlockSpec((16,), lambda i: (0,)),
        ],
        out_specs=pl.BlockSpec((bn, 16), lambda i: (i, 0)),
        out_shape=jax.ShapeDtypeStruct((npad, 16), jnp.float32),
    )(s1, s1, u, dinv, W1, b1)

    s2 = _sc_edge(src_rows, dst_rows, p, npad, 16)

    out = pl.pallas_call(
        _l2_pool_body,
        grid=(grid,),
        in_specs=[
            pl.BlockSpec((bn, 16), lambda i: (i, 0)),
            pl.BlockSpec((bn, 16), lambda i: (i + grid, 0)),
            pl.BlockSpec((bn, 16), lambda i: (i, 0)),
            pl.BlockSpec((bn, 1), lambda i: (i, 0)),
            pl.BlockSpec((bn, 1), lambda i: (i, 0)),
            pl.BlockSpec((16, 16), lambda i: (0, 0)),
            pl.BlockSpec((16,), lambda i: (0,)),
            pl.BlockSpec((16, 1), lambda i: (0, 0)),
            pl.BlockSpec((1,), lambda i: (0,)),
        ],
        out_specs=pl.BlockSpec((G_OUT, 1), lambda i: (0, 0)),
        out_shape=jax.ShapeDtypeStruct((G_OUT, 1), jnp.float32),
        scratch_shapes=[pltpu.VMEM((G_OUT, 17), jnp.float32)],
    )(s2, s2, p, dinv, batch_pad, W2, b2, Wlin, blin)

    return out


# speedup vs baseline: 122.3243x; 1.5888x over previous
import functools

import jax
import jax.numpy as jnp
import numpy as np
from jax import lax
from jax.experimental import pallas as pl
from jax.experimental.pallas import tpu as pltpu
from jax.experimental.pallas import tpu_sc as plsc

NC = 2
NS = 16
NW = NC * NS
LN = 128
BL = 8
G_OUT = 128
GPAD = 160


def _worker_rows(wid, total_rows):
    ngrp = total_rows // 8
    base = ngrp // NW
    rem = ngrp % NW
    grps = jnp.where(wid < rem, base + 1, base)
    grp0 = jnp.where(wid < rem, wid * (base + 1),
                     rem * (base + 1) + (wid - rem) * base)
    return grp0 * 8, grps * 8


def _zero_init(zeros_hbm, acc_sh, bnc_v, s):
    npad = acc_sh.shape[0]
    sl = npad // NS
    bc = zeros_hbm.shape[0]
    pltpu.sync_copy(zeros_hbm, bnc_v)
    off = pl.multiple_of(s * sl, 8)
    for q in range(sl // bc):
        pltpu.sync_copy(bnc_v, acc_sh.at[pl.ds(pl.multiple_of(off + q * bc, 8), bc)])
    return off, sl, bc


def _drain(acc_sh, out_hbm, bnc_v, c, s, off, sl, bc):
    npad = acc_sh.shape[0]
    for q in range(sl // bc):
        qo = pl.multiple_of(off + q * bc, 8)
        oo = pl.multiple_of(c * npad + s * sl + q * bc, 8)
        pltpu.sync_copy(acc_sh.at[pl.ds(qo, bc)], bnc_v)
        pltpu.sync_copy(bnc_v, out_hbm.at[pl.ds(oo, bc)])


def _deg_body(dst_hbm, batch_hbm, zeros_hbm, out_hbm, cnt_hbm,
              acc_sh, cnt_sh, idx_v, ones_v, bnc_v, sem_i, sem_s):
    c = lax.axis_index("c")
    s = lax.axis_index("s")
    wid = s * NC + c

    @pl.loop(0, LN)
    def _(i):
        ones_v[i, :] = jnp.ones((16,), jnp.float32)

    off, sl, bc = _zero_init(zeros_hbm, acc_sh, bnc_v, s)

    @pl.when(s == 0)
    def _():
        pltpu.sync_copy(bnc_v.at[pl.ds(0, GPAD)], cnt_sh)
    plsc.subcore_barrier()

    row0, rows = _worker_rows(wid, dst_hbm.shape[0])
    nb = rows // BL
    pltpu.async_copy(dst_hbm.at[pl.ds(row0, BL)], idx_v.at[0], sem_i)

    @pl.loop(0, nb)
    def _(g):
        par = g % 2
        r0 = pl.multiple_of(row0 + g * BL, 8)
        pltpu.make_async_copy(dst_hbm.at[pl.ds(r0, BL)], idx_v.at[par], sem_i).wait()

        @pl.when(g + 1 < nb)
        def _():
            rn = pl.multiple_of(row0 + (g + 1) * BL, 8)
            pltpu.async_copy(dst_hbm.at[pl.ds(rn, BL)], idx_v.at[1 - par], sem_i)

        for j in range(BL):
            pltpu.async_copy(ones_v, acc_sh.at[idx_v.at[par, j]], sem_s, add=True)
        for j in range(BL):
            pltpu.make_async_copy(ones_v, acc_sh.at[idx_v.at[par, j]], sem_s).wait()

    brow0, brows = _worker_rows(wid, batch_hbm.shape[0])
    bnb = brows // BL

    @pl.loop(0, bnb)
    def _(g):
        par = g % 2
        r0 = pl.multiple_of(brow0 + g * BL, 8)
        pltpu.sync_copy(batch_hbm.at[pl.ds(r0, BL)], idx_v.at[par])
        for j in range(BL):
            pltpu.async_copy(ones_v, cnt_sh.at[idx_v.at[par, j]], sem_s, add=True)
        for j in range(BL):
            pltpu.make_async_copy(ones_v, cnt_sh.at[idx_v.at[par, j]], sem_s).wait()

    plsc.subcore_barrier()
    _drain(acc_sh, out_hbm, bnc_v, c, s, off, sl, bc)

    @pl.when(s == 0)
    def _():
        pltpu.sync_copy(cnt_sh, bnc_v.at[pl.ds(0, GPAD)])
        pltpu.sync_copy(bnc_v.at[pl.ds(0, GPAD)],
                        cnt_hbm.at[pl.ds(pl.multiple_of(c * GPAD, 8), GPAD)])


def _edge_body(src_hbm, dst_hbm, tab_hbm, zeros_hbm, out_hbm,
               acc_sh, sidx_v, didx_v, val_v, bnc_v, sem_i, sem_g, sem_s):
    c = lax.axis_index("c")
    s = lax.axis_index("s")
    wid = s * NC + c

    off, sl, bc = _zero_init(zeros_hbm, acc_sh, bnc_v, s)
    plsc.subcore_barrier()

    row0, rows = _worker_rows(wid, src_hbm.shape[0])
    nb = rows // BL

    pltpu.async_copy(src_hbm.at[pl.ds(row0, BL)], sidx_v.at[0], sem_i)
    pltpu.async_copy(dst_hbm.at[pl.ds(row0, BL)], didx_v.at[0], sem_i)

    @pl.loop(0, nb)
    def _(g):
        par = g % 2
        r0 = pl.multiple_of(row0 + g * BL, 8)
        pltpu.make_async_copy(src_hbm.at[pl.ds(r0, BL)], sidx_v.at[par], sem_i).wait()
        pltpu.make_async_copy(dst_hbm.at[pl.ds(r0, BL)], didx_v.at[par], sem_i).wait()
        for j in range(BL):
            pltpu.async_copy(tab_hbm.at[sidx_v.at[par, j]], val_v.at[j], sem_g)

        @pl.when(g + 1 < nb)
        def _():
            rn = pl.multiple_of(row0 + (g + 1) * BL, 8)
            pltpu.async_copy(src_hbm.at[pl.ds(rn, BL)], sidx_v.at[1 - par], sem_i)
            pltpu.async_copy(dst_hbm.at[pl.ds(rn, BL)], didx_v.at[1 - par], sem_i)

        for j in range(BL):
            pltpu.make_async_copy(tab_hbm.at[sidx_v.at[par, j]], val_v.at[j], sem_g).wait()
            pltpu.async_copy(val_v.at[j], acc_sh.at[didx_v.at[par, j]], sem_s, add=True)
        for j in range(BL):
            pltpu.make_async_copy(val_v.at[j], acc_sh.at[didx_v.at[par, j]], sem_s).wait()

    plsc.subcore_barrier()
    _drain(acc_sh, out_hbm, bnc_v, c, s, off, sl, bc)


def _pool_body(h2_hbm, batch_hbm, zeros_hbm, out_hbm,
               acc_sh, bidx_v, hval_v, bnc_v, sem_s):
    c = lax.axis_index("c")
    s = lax.axis_index("s")
    wid = s * NC + c

    pltpu.sync_copy(zeros_hbm, bnc_v)

    @pl.when(s == 0)
    def _():
        pltpu.sync_copy(bnc_v, acc_sh)
    plsc.subcore_barrier()

    row0, rows = _worker_rows(wid, batch_hbm.shape[0])
    nb = rows // BL

    @pl.loop(0, nb)
    def _(g):
        r0 = pl.multiple_of(row0 + g * BL, 8)
        pltpu.sync_copy(batch_hbm.at[pl.ds(r0, BL)], bidx_v)
        h0 = pl.multiple_of((row0 + g * BL) * LN, 8)
        pltpu.sync_copy(h2_hbm.at[pl.ds(h0, BL * LN)], hval_v)
        for j in range(BL):
            pltpu.async_copy(hval_v.at[pl.ds(j * LN, LN)],
                             acc_sh.at[bidx_v.at[j]], sem_s, add=True)
        for j in range(BL):
            pltpu.make_async_copy(hval_v.at[pl.ds(j * LN, LN)],
                                  acc_sh.at[bidx_v.at[j]], sem_s).wait()

    plsc.subcore_barrier()

    @pl.when(s == 0)
    def _():
        pltpu.sync_copy(acc_sh, bnc_v)
        pltpu.sync_copy(bnc_v,
                        out_hbm.at[pl.ds(pl.multiple_of(c * GPAD, 8), GPAD)])


def _sc_deg(dst_rows, batch_rows, npad):
    sl = npad // NS
    bc = 256
    mesh = plsc.VectorSubcoreMesh(core_axis_name="c", subcore_axis_name="s")
    zeros = jnp.zeros((bc, 16), jnp.float32)
    return pl.kernel(
        _deg_body,
        out_type=(jax.ShapeDtypeStruct((NC * npad, 16), jnp.float32),
                  jax.ShapeDtypeStruct((NC * GPAD, 16), jnp.float32)),
        mesh=mesh,
        compiler_params=pltpu.CompilerParams(use_tc_tiling_on_sc=False),
        scratch_types=[
            pltpu.VMEM_SHARED((npad, 16), jnp.float32),
            pltpu.VMEM_SHARED((GPAD, 16), jnp.float32),
            pltpu.VMEM((2, BL, LN), jnp.int32),
            pltpu.VMEM((LN, 16), jnp.float32),
            pltpu.VMEM((bc, 16), jnp.float32),
            pltpu.SemaphoreType.DMA,
            pltpu.SemaphoreType.DMA,
        ],
    )(dst_rows, batch_rows, zeros)


def _sc_edge(src_rows, dst_rows, table, npad):
    sl = npad // NS
    bc = 256
    mesh = plsc.VectorSubcoreMesh(core_axis_name="c", subcore_axis_name="s")
    zeros = jnp.zeros((bc, 16), jnp.float32)
    return pl.kernel(
        _edge_body,
        out_type=jax.ShapeDtypeStruct((NC * npad, 16), jnp.float32),
        mesh=mesh,
        compiler_params=pltpu.CompilerParams(use_tc_tiling_on_sc=False),
        scratch_types=[
            pltpu.VMEM_SHARED((npad, 16), jnp.float32),
            pltpu.VMEM((2, BL, LN), jnp.int32),
            pltpu.VMEM((2, BL, LN), jnp.int32),
            pltpu.VMEM((BL, LN, 16), jnp.float32),
            pltpu.VMEM((bc, 16), jnp.float32),
            pltpu.SemaphoreType.DMA,
            pltpu.SemaphoreType.DMA,
            pltpu.SemaphoreType.DMA,
        ],
    )(src_rows, dst_rows, table, zeros)


def _sc_pool(h2, batch_rows, npad):
    mesh = plsc.VectorSubcoreMesh(core_axis_name="c", subcore_axis_name="s")
    zeros = jnp.zeros((GPAD, 16), jnp.float32)
    return pl.kernel(
        _pool_body,
        out_type=jax.ShapeDtypeStruct((NC * GPAD, 16), jnp.float32),
        mesh=mesh,
        compiler_params=pltpu.CompilerParams(use_tc_tiling_on_sc=False),
        scratch_types=[
            pltpu.VMEM_SHARED((GPAD, 16), jnp.float32),
            pltpu.VMEM((BL, LN), jnp.int32),
            pltpu.VMEM((BL * LN, 16), jnp.float32),
            pltpu.VMEM((GPAD, 16), jnp.float32),
            pltpu.SemaphoreType.DMA,
        ],
    )(h2, batch_rows, zeros)


def _prep_body(deg0_ref, deg1_ref, x_ref, rx_ref, eb_ref, dinv_ref, u_ref):
    dinv = lax.rsqrt(deg0_ref[...] + deg1_ref[...] + 1.0)
    dinv_ref[...] = dinv
    xb = x_ref[...]
    cs = [jnp.dot(xb, rx_ref[m * 128:(m + 1) * 128, :],
                  preferred_element_type=jnp.float32) for m in range(8)]
    x16 = jnp.dot(eb_ref[...], jnp.concatenate(cs, axis=0),
                  preferred_element_type=jnp.float32)
    u_ref[...] = x16 * dinv


def _layer_body(s0_ref, s1_ref, u_ref, dinv_ref, bd_ref, bt_ref, p_ref, *,
                scale_out):
    m = s0_ref[...] + s1_ref[...] + u_ref[...]
    z = jnp.dot(m, bd_ref[...], preferred_element_type=jnp.float32)
    dinv = dinv_ref[...]
    h = jnp.tanh(dinv * z + bt_ref[...][None, :])
    p_ref[...] = h * dinv if scale_out else h


def _head_body(pool_ref, cnt_ref, wlin_ref, blin_ref, out_ref):
    ps = pool_ref[0:G_OUT, :] + pool_ref[GPAD:GPAD + G_OUT, :]
    cs = cnt_ref[0:G_OUT, 0:1] + cnt_ref[GPAD:GPAD + G_OUT, 0:1]
    pooled = ps / jnp.maximum(cs, 1.0)
    out_ref[...] = (jnp.dot(pooled, wlin_ref[...],
                            preferred_element_type=jnp.float32)
                    + blin_ref[...][None, :])


def kernel(x, edge_index, batch, W1, b1, W2, b2, Wlin, blin):
    n = x.shape[0]
    e = edge_index.shape[1]
    assert e % LN == 0
    r = e // LN
    assert r % 8 == 0
    npad = 102400
    assert n <= npad
    r16 = npad * 16 // 128
    nrx = npad * 2 // 128
    grid = 20
    b16 = r16 // grid
    bx = nrx // grid

    src_rows = edge_index[0].reshape(r, LN)
    dst_rows = edge_index[1].reshape(r, LN)
    x_pk = jnp.concatenate(
        [x.reshape(-1), jnp.zeros((2 * (npad - n),), x.dtype)]).reshape(nrx, 128)
    batch_pad = jnp.concatenate(
        [batch,
         (G_OUT + jnp.arange(npad - n, dtype=batch.dtype) % (GPAD - G_OUT))])
    batch_rows = batch_pad.reshape(npad // LN, LN)

    rx_np = np.zeros((8, 128, 128), np.float32)
    for m_ in range(8):
        for k_ in range(8):
            for f_ in range(2):
                rx_np[m_, 16 * m_ + 2 * k_ + f_, 16 * k_ + f_] = 1.0
    rx = jnp.asarray(rx_np.reshape(1024, 128))
    eb_np = np.zeros((8 * bx, 8 * bx), np.float32)
    for q_ in range(bx):
        for m_ in range(8):
            eb_np[8 * q_ + m_, m_ * bx + q_] = 1.0
    eb = jnp.asarray(eb_np)

    bd1 = jnp.kron(jnp.eye(8, dtype=jnp.float32),
                   jnp.pad(W1, ((0, 14), (0, 0))))
    bd2 = jnp.kron(jnp.eye(8, dtype=jnp.float32), W2)
    b1t = jnp.tile(b1, 8)
    b2t = jnp.tile(b2, 8)

    deg16, cnt = _sc_deg(dst_rows, batch_rows, npad)
    deg_pk = deg16.reshape(NC * r16, 128)

    dinv16, u_pk = pl.pallas_call(
        _prep_body,
        grid=(grid,),
        in_specs=[
            pl.BlockSpec((b16, 128), lambda i: (i, 0)),
            pl.BlockSpec((b16, 128), lambda i: (i + grid, 0)),
            pl.BlockSpec((bx, 128), lambda i: (i, 0)),
            pl.BlockSpec((1024, 128), lambda i: (0, 0)),
            pl.BlockSpec((b16, bx * 8), lambda i: (0, 0)),
        ],
        out_specs=[
            pl.BlockSpec((b16, 128), lambda i: (i, 0)),
            pl.BlockSpec((b16, 128), lambda i: (i, 0)),
        ],
        out_shape=[
            jax.ShapeDtypeStruct((r16, 128), jnp.float32),
            jax.ShapeDtypeStruct((r16, 128), jnp.float32),
        ],
    )(deg_pk, deg_pk, x_pk, rx, eb)

    s1 = _sc_edge(src_rows, dst_rows, u_pk.reshape(npad, 16), npad)

    layer1 = functools.partial(_layer_body, scale_out=True)
    p_pk = pl.pallas_call(
        layer1,
        grid=(grid,),
        in_specs=[
            pl.BlockSpec((b16, 128), lambda i: (i, 0)),
            pl.BlockSpec((b16, 128), lambda i: (i + grid, 0)),
            pl.BlockSpec((b16, 128), lambda i: (i, 0)),
            pl.BlockSpec((b16, 128), lambda i: (i, 0)),
            pl.BlockSpec((128, 128), lambda i: (0, 0)),
            pl.BlockSpec((128,), lambda i: (0,)),
        ],
        out_specs=pl.BlockSpec((b16, 128), lambda i: (i, 0)),
        out_shape=jax.ShapeDtypeStruct((r16, 128), jnp.float32),
    )(s1.reshape(NC * r16, 128), s1.reshape(NC * r16, 128), u_pk, dinv16,
      bd1, b1t)

    s2 = _sc_edge(src_rows, dst_rows, p_pk.reshape(npad, 16), npad)

    layer2 = functools.partial(_layer_body, scale_out=False)
    h2_pk = pl.pallas_call(
        layer2,
        grid=(grid,),
        in_specs=[
            pl.BlockSpec((b16, 128), lambda i: (i, 0)),
            pl.BlockSpec((b16, 128), lambda i: (i + grid, 0)),
            pl.BlockSpec((b16, 128), lambda i: (i, 0)),
            pl.BlockSpec((b16, 128), lambda i: (i, 0)),
            pl.BlockSpec((128, 128), lambda i: (0, 0)),
            pl.BlockSpec((128,), lambda i: (0,)),
        ],
        out_specs=pl.BlockSpec((b16, 128), lambda i: (i, 0)),
        out_shape=jax.ShapeDtypeStruct((r16, 128), jnp.float32),
    )(s2.reshape(NC * r16, 128), s2.reshape(NC * r16, 128), p_pk, dinv16,
      bd2, b2t)

    pooled = _sc_pool(h2_pk.reshape(npad, 16), batch_rows, npad)

    out = pl.pallas_call(
        _head_body,
        grid=(1,),
        in_specs=[
            pl.BlockSpec((NC * GPAD, 16), lambda i: (0, 0)),
            pl.BlockSpec((NC * GPAD, 16), lambda i: (0, 0)),
            pl.BlockSpec((16, 1), lambda i: (0, 0)),
            pl.BlockSpec((1,), lambda i: (0,)),
        ],
        out_specs=pl.BlockSpec((G_OUT, 1), lambda i: (0, 0)),
        out_shape=jax.ShapeDtypeStruct((G_OUT, 1), jnp.float32),
    )(pooled, cnt, Wlin, blin)

    return out
